# Initial kernel scaffold; baseline (speedup 1.0000x reference)
import jax
import jax.numpy as jnp
from jax.experimental import pallas as pl


def kernel(pos, edge_index, W1, a_src1, a_dst1, b1, gamma1, beta1, W2, a_src2, a_dst2, b2, gamma2, beta2, W3, a_src3, a_dst3, b3, gamma3, beta3, W4, a_src4, a_dst4, b4, gamma4, beta4, lin1_W, lin1_b, lin2_W, lin2_b, lin3_W, lin3_b):
    raise NotImplementedError("write your pallas kernel here")


# speedup vs baseline: 1.7996x; 1.7996x over previous
import jax
import jax.numpy as jnp
from jax import lax
from jax.experimental import pallas as pl

N = 10000
E = 320000
DH = 128
DOUT = 40

NC = 2
NS = 16
HD = DH // NC
EPT = E // NS
CH = 80
NCHUNK = EPT // CH
WT = 10
RPT = N // WT
ZR = 40

_f32 = jnp.float32


def _leaky(x):
    return jnp.maximum(x, 0.2 * x)


def _emit_h(h, asv, adv, h2_ref, asn_ref, adn_ref, cv_ref):
    h2_ref[...] = h
    asn = jnp.sum(h * asv, axis=1, keepdims=True)
    adn = jnp.sum(h * adv, axis=1, keepdims=True)
    asn_ref[...] = asn
    adn_ref[...] = adn
    c = _leaky(jnp.max(asn) + jnp.max(adn))
    cv_ref[...] = jnp.full((1, 16), c, dtype=_f32)


def _tc_head_body(pos_ref, w_ref, asv_ref, adv_ref,
                  h2_ref, asn_ref, adn_ref, cv_ref):
    h = jnp.dot(pos_ref[...], w_ref[...], preferred_element_type=_f32)
    _emit_h(h, asv_ref[...], adv_ref[...], h2_ref, asn_ref, adn_ref, cv_ref)


def _combine(acc_ref, zp_ref, h2_ref, asn_ref, adn_ref, cv_ref,
             b_ref, g_ref, be_ref):
    c = cv_ref[0, 0]
    t = asn_ref[...] + adn_ref[...]
    exloop = jnp.exp(_leaky(t) - c)
    z = jnp.sum(zp_ref[...], axis=0)[:, None] + exloop
    h = h2_ref[...]
    raw = jnp.concatenate([acc_ref[:N, :], acc_ref[N:, :]], axis=1)
    raw = raw + exloop * h
    y = jnp.maximum(raw / z + b_ref[...], 0.0)
    mu = jnp.mean(y, axis=0, keepdims=True)
    var = jnp.mean(jnp.square(y - mu), axis=0, keepdims=True)
    return (y - mu) * lax.rsqrt(var + 1e-5) * g_ref[...] + be_ref[...]


def _tc_mid_body(acc_ref, zp_ref, h2_ref, asn_ref, adn_ref, cv_ref,
                 b_ref, g_ref, be_ref, w_ref, asv_ref, adv_ref,
                 o_h2_ref, o_asn_ref, o_adn_ref, o_cv_ref):
    xn = _combine(acc_ref, zp_ref, h2_ref, asn_ref, adn_ref, cv_ref,
                  b_ref, g_ref, be_ref)
    h = jnp.dot(xn, w_ref[...], preferred_element_type=_f32)
    _emit_h(h, asv_ref[...], adv_ref[...], o_h2_ref, o_asn_ref, o_adn_ref,
            o_cv_ref)


def _tc_tail_body(acc_ref, zp_ref, h2_ref, asn_ref, adn_ref, cv_ref,
                  b_ref, g_ref, be_ref,
                  l1w_ref, l1b_ref, l2w_ref, l2b_ref, l3w_ref, l3b_ref,
                  out_ref):
    xn = _combine(acc_ref, zp_ref, h2_ref, asn_ref, adn_ref, cv_ref,
                  b_ref, g_ref, be_ref)
    x = jnp.dot(xn, l1w_ref[...], preferred_element_type=_f32) + l1b_ref[...]
    x = jnp.dot(jnp.maximum(x, 0.0), l2w_ref[...],
                preferred_element_type=_f32) + l2b_ref[...]
    x = jnp.dot(x, l3w_ref[...], preferred_element_type=_f32) + l3b_ref[...]
    m = jnp.max(x, axis=1, keepdims=True)
    lse = jnp.log(jnp.sum(jnp.exp(x - m), axis=1, keepdims=True)) + m
    out_ref[...] = x - lse


_h_out_shapes = [
    jax.ShapeDtypeStruct((N, DH), _f32),
    jax.ShapeDtypeStruct((N, 1), _f32),
    jax.ShapeDtypeStruct((N, 1), _f32),
    jax.ShapeDtypeStruct((1, 16), _f32),
]

_tc_head = pl.pallas_call(_tc_head_body, out_shape=_h_out_shapes)
_tc_mid = pl.pallas_call(_tc_mid_body, out_shape=_h_out_shapes)
_tc_tail = pl.pallas_call(
    _tc_tail_body, out_shape=[jax.ShapeDtypeStruct((N, DOUT), _f32)])


def _edge_phase(src, dst, h, asn, adn, cv):
    c = cv[0]
    t = asn[src] + adn[dst]
    ex = jnp.exp(jnp.maximum(t, 0.2 * t) - c)
    z = jax.ops.segment_sum(ex, dst, num_segments=N)
    s = jax.ops.segment_sum(h[src] * ex[:, None], dst, num_segments=N)
    acc = jnp.concatenate([s[:, :HD], s[:, HD:]], axis=0)
    zp = jnp.concatenate([z, jnp.zeros(((NS - 1) * N,), _f32)])
    return acc, zp


def kernel(pos, edge_index,
           W1, a_src1, a_dst1, b1, gamma1, beta1,
           W2, a_src2, a_dst2, b2, gamma2, beta2,
           W3, a_src3, a_dst3, b3, gamma3, beta3,
           W4, a_src4, a_dst4, b4, gamma4, beta4,
           lin1_W, lin1_b, lin2_W, lin2_b, lin3_W, lin3_b):
    src = edge_index[0]
    dst = edge_index[1]

    h, asn, adn, cv = _tc_head(pos, W1, a_src1, a_dst1)

    layer_params = [
        (b1, gamma1, beta1, W2, a_src2, a_dst2),
        (b2, gamma2, beta2, W3, a_src3, a_dst3),
        (b3, gamma3, beta3, W4, a_src4, a_dst4),
    ]
    for b, g, be, w_next, as_next, ad_next in layer_params:
        acc, zp = _edge_phase(src, dst, h, asn.reshape(N), adn.reshape(N),
                              cv.reshape(16))
        h, asn, adn, cv = _tc_mid(acc, zp.reshape(NS, N), h, asn, adn, cv,
                                  b, g, be, w_next, as_next, ad_next)

    acc, zp = _edge_phase(src, dst, h, asn.reshape(N), adn.reshape(N),
                          cv.reshape(16))
    (out,) = _tc_tail(acc, zp.reshape(NS, N), h, asn, adn, cv,
                      b4, gamma4, beta4,
                      lin1_W, lin1_b, lin2_W, lin2_b, lin3_W, lin3_b)
    return out
